# baseline (speedup 1.0000x reference)
import jax
import jax.numpy as jnp
from jax import lax
from jax.experimental import pallas as pl
from jax.experimental.pallas import tpu as pltpu
from jax.experimental.pallas import tpu_sc as plsc

B = 4096
L = 200
D = 32
NC = 2
NS = 16
NW = NC * NS
BW = B // NW
LPB = 4
NB = L // LPB


def _emb_body(wi_hbm, ti_hbm, wt_hbm, tt_hbm, out_hbm,
              idx_w, idx_t, g_w, g_t, panel, sem, semw):
    wid = lax.axis_index("s") * NC + lax.axis_index("c")
    b0 = wid * BW
    iota = lax.iota(jnp.int32, 16)

    def step(nb, carry):
        l0 = nb * LPB
        pltpu.sync_copy(wi_hbm.at[pl.ds(l0, LPB), pl.ds(b0, BW)], idx_w)
        pltpu.sync_copy(ti_hbm.at[pl.ds(l0, LPB), pl.ds(b0, BW)], idx_t)
        gathers = []
        for li in range(LPB):
            gathers.append(pltpu.async_copy(
                wt_hbm.at[idx_w.at[li]], g_w.at[li], sem))
            gathers.append(pltpu.async_copy(
                tt_hbm.at[idx_t.at[li]], g_t.at[li], sem))
        for h in gathers:
            h.wait()

        def trow(r, carry2):
            rr = jnp.full((16,), r, dtype=jnp.int32)
            for li in range(LPB):
                lvec = jnp.full((16,), li, dtype=jnp.int32)
                for h in range(2):
                    vw = g_w[li, r, pl.ds(16 * h, 16)]
                    plsc.store_scatter(panel, [lvec, iota + 16 * h, rr], vw)
                    vt = g_t[li, r, pl.ds(16 * h, 16)]
                    plsc.store_scatter(panel, [lvec, iota + (D + 16 * h), rr], vt)
            return carry2

        lax.fori_loop(0, BW, trow, 0)
        pltpu.async_copy(
            panel, out_hbm.at[pl.ds(l0, LPB), :, pl.ds(b0, BW)], semw).wait()
        return carry

    lax.fori_loop(0, NB, step, 0)


def kernel(words_token_ids, tags_token_ids, words_table, tags_table):
    wi = words_token_ids.T
    ti = tags_token_ids.T
    mesh = plsc.VectorSubcoreMesh(core_axis_name="c", subcore_axis_name="s")
    out = pl.kernel(
        _emb_body,
        mesh=mesh,
        compiler_params=pltpu.CompilerParams(
            use_tc_tiling_on_sc=False, needs_layout_passes=False),
        out_type=jax.ShapeDtypeStruct((L, 2 * D, B), jnp.float32),
        scratch_types=[
            pltpu.VMEM((LPB, BW), jnp.int32),
            pltpu.VMEM((LPB, BW), jnp.int32),
            pltpu.VMEM((LPB, BW, D), jnp.float32),
            pltpu.VMEM((LPB, BW, D), jnp.float32),
            pltpu.VMEM((LPB, 2 * D, BW), jnp.float32),
            pltpu.SemaphoreType.DMA,
            pltpu.SemaphoreType.DMA,
        ],
    )(wi, ti, words_table, tags_table)
    return out.transpose(2, 0, 1)


# speedup vs baseline: 1.1932x; 1.1932x over previous
import jax
import jax.numpy as jnp
from jax import lax
from jax.experimental import pallas as pl
from jax.experimental.pallas import tpu as pltpu
from jax.experimental.pallas import tpu_sc as plsc

B = 4096
L = 200
D = 32
NC = 2
NS = 16
NW = NC * NS
BW = B // NW
LPB = 2
NSTEP = L // LPB
GR = LPB * BW
PROWS = LPB * 8
PCOLS = 8 * BW
OUT_R = L * 8
OUT_C = 32 * 8 * BW


def _emb_body(wi_hbm, ti_hbm, wt_hbm, tt_hbm, out_hbm,
              idx_w, idx_t, g_w, g_t, panel,
              sgw0, sgw1, sgt0, sgt1, sw0, sw1):
    sem_gw = (sgw0, sgw1)
    sem_gt = (sgt0, sgt1)
    sem_w = (sw0, sw1)
    wid = lax.axis_index("s") * NC + lax.axis_index("c")
    b0 = wid * BW
    iota = lax.iota(jnp.int32, 16)

    livecs = []
    dtvecs = []
    divecs = []
    for li in range(LPB):
        for tt in range(2):
            for h in range(2):
                d = 32 * tt + 16 * h + iota
                livecs.append(jnp.broadcast_to(li, (16,)).astype(jnp.int32))
                dtvecs.append(d // 8)
                divecs.append(d % 8)

    def fire_gathers(nb, p):
        l0 = nb * LPB
        pltpu.sync_copy(wi_hbm.at[pl.ds(l0, LPB), pl.ds(b0, BW)], idx_w.at[p])
        pltpu.sync_copy(ti_hbm.at[pl.ds(l0, LPB), pl.ds(b0, BW)], idx_t.at[p])
        for li in range(LPB):
            pltpu.async_copy(
                wt_hbm.at[idx_w.at[p, li]],
                g_w.at[p, pl.ds(li * BW, BW), :], sem_gw[p])
            pltpu.async_copy(
                tt_hbm.at[idx_t.at[p, li]],
                g_t.at[p, pl.ds(li * BW, BW), :], sem_gt[p])

    def wait_gathers(p):
        for li in range(LPB):
            pltpu.make_async_copy(
                wt_hbm.at[pl.ds(0, BW)],
                g_w.at[p, pl.ds(li * BW, BW), :], sem_gw[p]).wait()
            pltpu.make_async_copy(
                tt_hbm.at[pl.ds(0, BW)],
                g_t.at[p, pl.ds(li * BW, BW), :], sem_gt[p]).wait()

    def out_slice(nb):
        return out_hbm.at[pl.ds(nb * LPB, LPB), :, wid, :, :]

    def wait_write(p):
        pltpu.make_async_copy(panel.at[p], out_slice(0), sem_w[p]).wait()

    fire_gathers(0, 0)
    fire_gathers(1, 1)

    def pair(hstep, carry):
        for p in range(2):
            nb = 2 * hstep + p
            wait_gathers(p)

            @pl.when(nb >= 2)
            def _():
                wait_write(p)

            def trow(r, carry2):
                rv = jnp.broadcast_to(r, (16,)).astype(jnp.int32)
                k = 0
                for li in range(LPB):
                    row = li * BW + r
                    for g_ref in (g_w, g_t):
                        for h in range(2):
                            v = g_ref[p, row, pl.ds(16 * h, 16)]
                            plsc.store_scatter(
                                panel.at[p],
                                [livecs[k], dtvecs[k], divecs[k], rv], v)
                            k += 1
                return carry2

            lax.fori_loop(0, BW, trow, 0)
            pltpu.async_copy(panel.at[p], out_slice(nb), sem_w[p])
            fire_gathers(lax.min(nb + 2, NSTEP - 1), p)
        return carry

    lax.fori_loop(0, NSTEP // 2, pair, 0)
    for p in range(2):
        wait_gathers(p)
        wait_write(p)


def kernel(words_token_ids, tags_token_ids, words_table, tags_table):
    wi = words_token_ids.T
    ti = tags_token_ids.T
    mesh = plsc.VectorSubcoreMesh(core_axis_name="c", subcore_axis_name="s")
    out = pl.kernel(
        _emb_body,
        mesh=mesh,
        compiler_params=pltpu.CompilerParams(
            use_tc_tiling_on_sc=False, needs_layout_passes=False),
        out_type=jax.ShapeDtypeStruct((L, 8, 32, 8, BW), jnp.float32),
        scratch_types=[
            pltpu.VMEM((2, LPB, BW), jnp.int32),
            pltpu.VMEM((2, LPB, BW), jnp.int32),
            pltpu.VMEM((2, GR, D), jnp.float32),
            pltpu.VMEM((2, GR, D), jnp.float32),
            pltpu.VMEM((2, LPB, 8, 8, BW), jnp.float32),
            pltpu.SemaphoreType.DMA,
            pltpu.SemaphoreType.DMA,
            pltpu.SemaphoreType.DMA,
            pltpu.SemaphoreType.DMA,
            pltpu.SemaphoreType.DMA,
            pltpu.SemaphoreType.DMA,
        ],
    )(wi, ti, words_table, tags_table)
    return out.transpose(2, 4, 0, 1, 3).reshape(B, L, 2 * D)


# speedup vs baseline: 1.4172x; 1.1877x over previous
import jax
import jax.numpy as jnp
from jax import lax
from jax.experimental import pallas as pl
from jax.experimental.pallas import tpu as pltpu
from jax.experimental.pallas import tpu_sc as plsc

B = 4096
L = 200
D = 32
NC = 2
NS = 16
NW = NC * NS
BW = B // NW
LPB = 2
NSTEP = L // LPB
GR = LPB * BW
PROWS = LPB * 8
PCOLS = 8 * BW
OUT_R = L * 8
OUT_C = 32 * 8 * BW


def _emb_body(wi_hbm, ti_hbm, wt_hbm, tt_hbm, out_hbm,
              idx_w, idx_t, g_w, g_t, panel,
              sgw0, sgw1, sgt0, sgt1, sw0, sw1):
    sem_gw = (sgw0, sgw1)
    sem_gt = (sgt0, sgt1)
    sem_w = (sw0, sw1)
    wid = lax.axis_index("s") * NC + lax.axis_index("c")
    b0 = wid * BW
    iota = lax.iota(jnp.int32, 16)

    rowvecs = []
    divecs = []
    for li in range(LPB):
        for tt in range(2):
            for h in range(2):
                d = 32 * tt + 16 * h + iota
                rowvecs.append(li * 8 + d // 8)
                divecs.append(d % 8)

    def fire_gathers(nb, p):
        l0 = nb * LPB
        pltpu.sync_copy(wi_hbm.at[pl.ds(l0, LPB), pl.ds(b0, BW)], idx_w.at[p])
        pltpu.sync_copy(ti_hbm.at[pl.ds(l0, LPB), pl.ds(b0, BW)], idx_t.at[p])
        for li in range(LPB):
            pltpu.async_copy(
                wt_hbm.at[idx_w.at[p, li]],
                g_w.at[p, pl.ds(li * BW, BW), :], sem_gw[p])
            pltpu.async_copy(
                tt_hbm.at[idx_t.at[p, li]],
                g_t.at[p, pl.ds(li * BW, BW), :], sem_gt[p])

    def wait_gathers(p):
        for li in range(LPB):
            pltpu.make_async_copy(
                wt_hbm.at[pl.ds(0, BW)],
                g_w.at[p, pl.ds(li * BW, BW), :], sem_gw[p]).wait()
            pltpu.make_async_copy(
                tt_hbm.at[pl.ds(0, BW)],
                g_t.at[p, pl.ds(li * BW, BW), :], sem_gt[p]).wait()

    def out_slice(l):
        return out_hbm.at[l, :, wid, :, :]

    def wait_write(p):
        for li in range(LPB):
            pltpu.make_async_copy(
                panel.at[p, pl.ds(li * 8, 8)], out_slice(0), sem_w[p]).wait()

    fire_gathers(0, 0)
    fire_gathers(1, 1)

    def pair(hstep, carry):
        for p in range(2):
            nb = 2 * hstep + p
            wait_gathers(p)

            @pl.when(nb >= 2)
            def _():
                wait_write(p)

            @plsc.parallel_loop(0, BW, unroll=8)
            def trow(r):
                rv = jnp.broadcast_to(r, (16,)).astype(jnp.int32)
                k = 0
                for li in range(LPB):
                    row = li * BW + r
                    for g_ref in (g_w, g_t):
                        for h in range(2):
                            v = g_ref[p, row, pl.ds(16 * h, 16)]
                            plsc.store_scatter(
                                panel.at[p],
                                [rowvecs[k], divecs[k], rv], v)
                            k += 1

            for li in range(LPB):
                pltpu.async_copy(
                    panel.at[p, pl.ds(li * 8, 8)],
                    out_slice(nb * LPB + li), sem_w[p])
            fire_gathers(lax.min(nb + 2, NSTEP - 1), p)
        return carry

    lax.fori_loop(0, NSTEP // 2, pair, 0)
    for p in range(2):
        wait_gathers(p)
        wait_write(p)


def kernel(words_token_ids, tags_token_ids, words_table, tags_table):
    wi = words_token_ids.T
    ti = tags_token_ids.T
    mesh = plsc.VectorSubcoreMesh(core_axis_name="c", subcore_axis_name="s")
    out = pl.kernel(
        _emb_body,
        mesh=mesh,
        compiler_params=pltpu.CompilerParams(
            use_tc_tiling_on_sc=False, needs_layout_passes=False),
        out_type=jax.ShapeDtypeStruct((L, 8, 32, 8, BW), jnp.float32),
        scratch_types=[
            pltpu.VMEM((2, LPB, BW), jnp.int32),
            pltpu.VMEM((2, LPB, BW), jnp.int32),
            pltpu.VMEM((2, GR, D), jnp.float32),
            pltpu.VMEM((2, GR, D), jnp.float32),
            pltpu.VMEM((2, LPB * 8, 8, BW), jnp.float32),
            pltpu.SemaphoreType.DMA,
            pltpu.SemaphoreType.DMA,
            pltpu.SemaphoreType.DMA,
            pltpu.SemaphoreType.DMA,
            pltpu.SemaphoreType.DMA,
            pltpu.SemaphoreType.DMA,
        ],
    )(wi, ti, words_table, tags_table)
    return out.transpose(2, 4, 0, 1, 3).reshape(B, L, 2 * D)
